# Initial kernel scaffold; baseline (speedup 1.0000x reference)
import jax
import jax.numpy as jnp
from jax import lax
from jax.experimental import pallas as pl
from jax.experimental.pallas import tpu as pltpu
from jax.experimental.pallas import tpu_sc as plsc

NUM_EMB = 1000000
D = 32
B_TOK = 16384
S_TOK = 50
B = B_TOK * S_TOK

NC = 2
NS = 16
NW = NC * NS
B_PER_W = B // NW

CHUNK = 128
N_CHUNKS = B_PER_W // CHUNK


def _body(idx_hbm, table_hbm, out_hbm, idx_v, rows_v, sem):
    wid = lax.axis_index("s") * NC + lax.axis_index("c")
    pltpu.sync_copy(idx_hbm.at[wid], idx_v)
    base = wid * B_PER_W

    def chunk_body(c, carry):
        pltpu.async_copy(table_hbm.at[idx_v.at[c]], rows_v, sem).wait()
        pltpu.sync_copy(rows_v, out_hbm.at[pl.ds(base + c * CHUNK, CHUNK)])
        return carry

    lax.fori_loop(0, N_CHUNKS, chunk_body, 0)


@jax.jit
def _gather(idx, weight):
    mesh = plsc.VectorSubcoreMesh(
        core_axis_name="c", subcore_axis_name="s", num_cores=NC,
        num_subcores=NS)
    f = pl.kernel(
        _body,
        out_type=jax.ShapeDtypeStruct((B, D), jnp.float32),
        mesh=mesh,
        scratch_types=[
            pltpu.VMEM((N_CHUNKS, CHUNK), jnp.int32),
            pltpu.VMEM((CHUNK, D), jnp.float32),
            pltpu.SemaphoreType.DMA,
        ],
    )
    return f(idx, weight)


def kernel(token_ids, weight):
    idx = token_ids.reshape(NW, N_CHUNKS, CHUNK).astype(jnp.int32)
    out = _gather(idx, weight)
    return out.reshape(B_TOK, S_TOK, D)


# speedup vs baseline: 1.0228x; 1.0228x over previous
import jax
import jax.numpy as jnp
from jax import lax
from jax.experimental import pallas as pl
from jax.experimental.pallas import tpu as pltpu
from jax.experimental.pallas import tpu_sc as plsc

NUM_EMB = 1000000
D = 32
B_TOK = 16384
S_TOK = 50
B = B_TOK * S_TOK

NC = 2
NS = 16
NW = NC * NS
B_PER_W = B // NW

CHUNK = 128
N_CHUNKS = B_PER_W // CHUNK


def _body(idx_hbm, table_hbm, out_hbm, idx_v, rows_v, sem):
    wid = lax.axis_index("s") * NC + lax.axis_index("c")
    pltpu.sync_copy(idx_hbm.at[wid], idx_v)
    base = wid * B_PER_W

    def chunk_body(c, carry):
        pltpu.async_copy(table_hbm.at[idx_v.at[c]], rows_v, sem).wait()
        pltpu.sync_copy(rows_v, out_hbm.at[pl.ds(base + c * CHUNK, CHUNK)])
        return carry

    lax.fori_loop(0, N_CHUNKS, chunk_body, 0)


@jax.jit
def _gather(idx, weight):
    mesh = plsc.VectorSubcoreMesh(
        core_axis_name="c", subcore_axis_name="s", num_cores=NC,
        num_subcores=NS)
    f = pl.kernel(
        _body,
        out_type=jax.ShapeDtypeStruct((B, D), jnp.float32),
        mesh=mesh,
        scratch_types=[
            pltpu.VMEM((N_CHUNKS, CHUNK), jnp.int32),
            pltpu.VMEM((CHUNK, D), jnp.float32),
            pltpu.SemaphoreType.DMA,
        ],
        compiler_params=pltpu.CompilerParams(use_tc_tiling_on_sc=False),
    )
    return f(idx, weight)


def kernel(token_ids, weight):
    idx = token_ids.reshape(NW, N_CHUNKS, CHUNK).astype(jnp.int32)
    out = _gather(idx, weight)
    return out.reshape(B_TOK, S_TOK, D)


# speedup vs baseline: 1.1144x; 1.0896x over previous
import jax
import jax.numpy as jnp
from jax import lax
from jax.experimental import pallas as pl
from jax.experimental.pallas import tpu as pltpu
from jax.experimental.pallas import tpu_sc as plsc

NUM_EMB = 1000000
D = 32
B_TOK = 16384
S_TOK = 50
B = B_TOK * S_TOK

NC = 2
NS = 16
NW = NC * NS
B_PER_W = B // NW

CHUNK = 128
N_CHUNKS = B_PER_W // CHUNK
NBUF = 8
LOOKAHEAD = 6


def _body(idx_hbm, table_hbm, out_hbm, idx_v, rows_v, *sems):
    gsem = sems[:NBUF]
    ssem = sems[NBUF:]
    wid = lax.axis_index("s") * NC + lax.axis_index("c")
    pltpu.sync_copy(idx_hbm.at[wid], idx_v)
    base = wid * B_PER_W

    def gather(c, b):
        pltpu.async_copy(table_hbm.at[idx_v.at[c]], rows_v.at[b], gsem[b])

    def gather_wait(b):
        pltpu.make_async_copy(
            table_hbm.at[idx_v.at[0]], rows_v.at[b], gsem[b]).wait()

    def scatter(c, b):
        pltpu.async_copy(
            rows_v.at[b], out_hbm.at[pl.ds(base + c * CHUNK, CHUNK)], ssem[b])

    def scatter_wait(b):
        pltpu.make_async_copy(
            rows_v.at[b], out_hbm.at[pl.ds(base, CHUNK)], ssem[b]).wait()

    for c in range(LOOKAHEAD):
        gather(c, c)

    def group(i, carry):
        for b in range(NBUF):
            c = i * NBUF + b
            gather_wait(b)
            scatter(c, b)
            c2 = c + LOOKAHEAD
            b2 = (b + LOOKAHEAD) % NBUF

            @pl.when(c2 < N_CHUNKS)
            def _():
                @pl.when(c2 >= NBUF)
                def _():
                    scatter_wait(b2)
                gather(c2, b2)
        return carry

    lax.fori_loop(0, N_CHUNKS // NBUF, group, 0)

    for b in range(NBUF):
        scatter_wait(b)


@jax.jit
def _gather(idx, weight):
    mesh = plsc.VectorSubcoreMesh(
        core_axis_name="c", subcore_axis_name="s", num_cores=NC,
        num_subcores=NS)
    f = pl.kernel(
        _body,
        out_type=jax.ShapeDtypeStruct((B, D), jnp.float32),
        mesh=mesh,
        scratch_types=[
            pltpu.VMEM((N_CHUNKS, CHUNK), jnp.int32),
            pltpu.VMEM((NBUF, CHUNK, D), jnp.float32),
        ] + [pltpu.SemaphoreType.DMA] * (2 * NBUF),
        compiler_params=pltpu.CompilerParams(use_tc_tiling_on_sc=False),
    )
    return f(idx, weight)


def kernel(token_ids, weight):
    idx = token_ids.reshape(NW, N_CHUNKS, CHUNK).astype(jnp.int32)
    out = _gather(idx, weight)
    return out.reshape(B_TOK, S_TOK, D)


# speedup vs baseline: 1.8079x; 1.6223x over previous
import jax
import jax.numpy as jnp
from jax import lax
from jax.experimental import pallas as pl
from jax.experimental.pallas import tpu as pltpu
from jax.experimental.pallas import tpu_sc as plsc

NUM_EMB = 1000000
D = 32
B_TOK = 16384
S_TOK = 50

NC = 2
NS = 16
NW = NC * NS
BATCH_PER_W = B_TOK // NW

NBATCH = 4
CHUNK = NBATCH * S_TOK
N_CHUNKS = BATCH_PER_W // NBATCH
NBUF = 8
LOOKAHEAD = 6


def _body(idx_hbm, table_hbm, out_hbm, idx_v, rows_v, *sems):
    gsem = sems[:NBUF]
    ssem = sems[NBUF:]
    wid = lax.axis_index("s") * NC + lax.axis_index("c")
    b0 = wid * BATCH_PER_W
    pltpu.sync_copy(idx_hbm.at[pl.ds(b0, BATCH_PER_W)], idx_v)

    def gather(c, b):
        for j in range(NBATCH):
            pltpu.async_copy(
                table_hbm.at[idx_v.at[c * NBATCH + j]],
                rows_v.at[b, j], gsem[b])

    def gather_wait(b):
        for j in range(NBATCH):
            pltpu.make_async_copy(
                table_hbm.at[idx_v.at[j]],
                rows_v.at[b, j], gsem[b]).wait()

    def scatter(c, b):
        pltpu.async_copy(
            rows_v.at[b],
            out_hbm.at[pl.ds(b0 + c * NBATCH, NBATCH)], ssem[b])

    def scatter_wait(b):
        pltpu.make_async_copy(
            rows_v.at[b],
            out_hbm.at[pl.ds(b0, NBATCH)], ssem[b]).wait()

    for c in range(LOOKAHEAD):
        gather(c, c)

    def group(i, carry):
        for b in range(NBUF):
            c = i * NBUF + b
            gather_wait(b)
            scatter(c, b)
            c2 = c + LOOKAHEAD
            b2 = (b + LOOKAHEAD) % NBUF

            @pl.when(c2 < N_CHUNKS)
            def _():
                @pl.when(c2 >= NBUF)
                def _():
                    scatter_wait(b2)
                gather(c2, b2)
        return carry

    lax.fori_loop(0, N_CHUNKS // NBUF, group, 0)

    for b in range(NBUF):
        scatter_wait(b)


@jax.jit
def _gather(idx, weight):
    mesh = plsc.VectorSubcoreMesh(
        core_axis_name="c", subcore_axis_name="s", num_cores=NC,
        num_subcores=NS)
    f = pl.kernel(
        _body,
        out_type=jax.ShapeDtypeStruct((B_TOK, S_TOK, D), jnp.float32),
        mesh=mesh,
        scratch_types=[
            pltpu.VMEM((BATCH_PER_W, S_TOK), jnp.int32),
            pltpu.VMEM((NBUF, NBATCH, S_TOK, D), jnp.float32),
        ] + [pltpu.SemaphoreType.DMA] * (2 * NBUF),
        compiler_params=pltpu.CompilerParams(use_tc_tiling_on_sc=False),
    )
    return f(idx, weight)


def kernel(token_ids, weight):
    return _gather(token_ids.astype(jnp.int32), weight)


# speedup vs baseline: 2.5445x; 1.4074x over previous
import jax
import jax.numpy as jnp
from jax import lax
from jax.experimental import pallas as pl
from jax.experimental.pallas import tpu as pltpu
from jax.experimental.pallas import tpu_sc as plsc

NUM_EMB = 1000000
D = 32
B_TOK = 16384
S_TOK = 50

NC = 2
NS = 16
NW = NC * NS
BATCH_PER_W = B_TOK // NW

NBATCH = 4
CHUNK = NBATCH * S_TOK
N_CHUNKS = BATCH_PER_W // NBATCH
NBUF = 8
LOOKAHEAD = 6


def _body(idx_hbm, table_hbm, out_hbm, idx_v, rows_v, *sems):
    gsem = sems[:NBUF]
    ssem = sems[NBUF:]
    wid = lax.axis_index("s") * NC + lax.axis_index("c")
    b0 = wid * BATCH_PER_W
    pltpu.sync_copy(idx_hbm.at[pl.ds(b0, BATCH_PER_W)], idx_v)

    def gather(c, b):
        for j in range(NBATCH):
            pltpu.async_copy(
                table_hbm.at[idx_v.at[c * NBATCH + j]],
                rows_v.at[b, j], gsem[b])

    def gather_wait(b):
        for j in range(NBATCH):
            pltpu.make_async_copy(
                table_hbm.at[idx_v.at[j]],
                rows_v.at[b, j], gsem[b]).wait()

    def scatter(c, b):
        pltpu.async_copy(
            rows_v.at[b],
            out_hbm.at[pl.ds(b0 + c * NBATCH, NBATCH), 0:S_TOK, 0:D],
            ssem[b])

    def scatter_wait(b):
        pltpu.make_async_copy(
            rows_v.at[b],
            out_hbm.at[pl.ds(b0, NBATCH), 0:S_TOK, 0:D], ssem[b]).wait()

    for c in range(LOOKAHEAD):
        gather(c, c)

    def group(i, carry):
        for b in range(NBUF):
            c = i * NBUF + b
            gather_wait(b)
            scatter(c, b)
            c2 = c + LOOKAHEAD
            b2 = (b + LOOKAHEAD) % NBUF

            @pl.when(c2 < N_CHUNKS)
            def _():
                @pl.when(c2 >= NBUF)
                def _():
                    scatter_wait(b2)
                gather(c2, b2)
        return carry

    lax.fori_loop(0, N_CHUNKS // NBUF, group, 0)

    for b in range(NBUF):
        scatter_wait(b)


S_PAD = 56
D_PAD = 128


def _final_body(zpad_hbm, out_hbm):
    wid = lax.axis_index("s") * NC + lax.axis_index("c")
    b0 = wid * BATCH_PER_W

    def chunk_body(c, carry):
        bb = b0 + c * NBATCH
        pltpu.sync_copy(
            zpad_hbm.at[pl.ds(bb, NBATCH), 0:S_TOK, 0:D],
            out_hbm.at[pl.ds(bb, NBATCH)])
        return carry

    lax.fori_loop(0, N_CHUNKS, chunk_body, 0)


def _mesh():
    return plsc.VectorSubcoreMesh(
        core_axis_name="c", subcore_axis_name="s", num_cores=NC,
        num_subcores=NS)


@jax.jit
def _gather(idx, weight):
    f = pl.kernel(
        _body,
        out_type=jax.ShapeDtypeStruct((B_TOK, S_PAD, D_PAD), jnp.float32),
        mesh=_mesh(),
        scratch_types=[
            pltpu.VMEM((BATCH_PER_W, S_TOK), jnp.int32),
            pltpu.VMEM((NBUF, NBATCH, S_TOK, D), jnp.float32),
        ] + [pltpu.SemaphoreType.DMA] * (2 * NBUF),
        compiler_params=pltpu.CompilerParams(use_tc_tiling_on_sc=False),
    )
    zpad = f(idx, weight)
    return lax.slice(zpad, (0, 0, 0), (B_TOK, S_TOK, D))


def kernel(token_ids, weight):
    return _gather(token_ids.astype(jnp.int32), weight)
